# baseline (speedup 1.0000x reference)
import jax
import jax.numpy as jnp
from jax.experimental import pallas as pl
from jax.experimental.pallas import tpu as pltpu

_NCHUNK = 4


def _dma_copy(pe_ref, out_ref, sems):
    rows = out_ref.shape[0]
    chunk = rows // _NCHUNK
    copies = [
        pltpu.make_async_copy(
            pe_ref.at[pl.ds(i * chunk, chunk)],
            out_ref.at[pl.ds(i * chunk, chunk)],
            sems.at[i],
        )
        for i in range(_NCHUNK)
    ]
    for c in copies:
        c.start()
    for c in copies:
        c.wait()


def kernel(x, pe):
    seq_len = x.shape[1]
    d = pe.shape[1]
    out = pl.pallas_call(
        _dma_copy,
        in_specs=[pl.BlockSpec(memory_space=pl.ANY)],
        out_specs=pl.BlockSpec(memory_space=pl.ANY),
        out_shape=jax.ShapeDtypeStruct((seq_len, d), pe.dtype),
        scratch_shapes=[pltpu.SemaphoreType.DMA((_NCHUNK,))],
    )(pe)
    return out[None]


# speedup vs baseline: 29.4035x; 29.4035x over previous
import jax
import jax.numpy as jnp
from jax.experimental import pallas as pl
from jax.experimental.pallas import tpu as pltpu

_BLOCK = 256


def _copy_block(pe_ref, out_ref):
    out_ref[...] = pe_ref[...]


def kernel(x, pe):
    seq_len = x.shape[1]
    d = pe.shape[1]
    out = pl.pallas_call(
        _copy_block,
        grid=(seq_len // _BLOCK,),
        in_specs=[pl.BlockSpec((_BLOCK, d), lambda i: (i, 0))],
        out_specs=pl.BlockSpec((_BLOCK, d), lambda i: (i, 0)),
        out_shape=jax.ShapeDtypeStruct((seq_len, d), pe.dtype),
        compiler_params=pltpu.CompilerParams(
            dimension_semantics=("arbitrary",),
        ),
    )(pe)
    return out[None]


# speedup vs baseline: 42.2018x; 1.4353x over previous
import jax
import jax.numpy as jnp
from jax.experimental import pallas as pl
from jax.experimental.pallas import tpu as pltpu

_BLOCK = 1024


def _copy_block(pe_ref, out_ref):
    out_ref[...] = pe_ref[...]


def kernel(x, pe):
    seq_len = x.shape[1]
    d = pe.shape[1]
    out = pl.pallas_call(
        _copy_block,
        grid=(seq_len // _BLOCK,),
        in_specs=[pl.BlockSpec((_BLOCK, d), lambda i: (i, 0))],
        out_specs=pl.BlockSpec((_BLOCK, d), lambda i: (i, 0)),
        out_shape=jax.ShapeDtypeStruct((seq_len, d), pe.dtype),
        compiler_params=pltpu.CompilerParams(
            dimension_semantics=("arbitrary",),
        ),
    )(pe)
    return out[None]


# speedup vs baseline: 47.5428x; 1.1266x over previous
import jax
import jax.numpy as jnp
from jax.experimental import pallas as pl
from jax.experimental.pallas import tpu as pltpu

_BLOCK = 2048


def _copy_block(pe_ref, out_ref):
    out_ref[...] = pe_ref[...]


def kernel(x, pe):
    seq_len = x.shape[1]
    d = pe.shape[1]
    out = pl.pallas_call(
        _copy_block,
        grid=(seq_len // _BLOCK,),
        in_specs=[pl.BlockSpec((_BLOCK, d), lambda i: (i, 0))],
        out_specs=pl.BlockSpec((_BLOCK, d), lambda i: (i, 0)),
        out_shape=jax.ShapeDtypeStruct((seq_len, d), pe.dtype),
        compiler_params=pltpu.CompilerParams(
            dimension_semantics=("arbitrary",),
        ),
    )(pe)
    return out[None]
